# Initial kernel scaffold; baseline (speedup 1.0000x reference)
import jax
import jax.numpy as jnp
from jax.experimental import pallas as pl


def kernel(x, Wg, bg, W1, b1, W2, b2):
    raise NotImplementedError("write your pallas kernel here")


# speedup vs baseline: 1.6288x; 1.6288x over previous
import functools

import jax
import jax.numpy as jnp
from jax.experimental import pallas as pl
from jax.experimental.pallas import tpu as pltpu

B = 4096
D = 1024
E = 8
K = 2
TM = 512


def _routing_body(x_ref, wg_ref, bg_ref, wfull_ref, cnt_ref, imp_ref,
                  ll_ref, il_ref):
    i = pl.program_id(0)
    x = x_ref[...]
    s = jax.lax.dot_general(
        x, wg_ref[...], (((1,), (1,)), ((), ())),
        preferred_element_type=jnp.float32) + bg_ref[...]
    ids = jax.lax.broadcasted_iota(jnp.int32, (TM, E), 1)
    m1 = jnp.max(s, axis=1, keepdims=True)
    a1 = jnp.min(jnp.where(s == m1, ids, E), axis=1, keepdims=True)
    s2 = jnp.where(ids == a1, -jnp.inf, s)
    m2 = jnp.max(s2, axis=1, keepdims=True)
    a2 = jnp.min(jnp.where(s2 == m2, ids, E), axis=1, keepdims=True)
    e21 = jnp.exp(m2 - m1)
    w1 = 1.0 / (1.0 + e21)
    w2 = e21 / (1.0 + e21)
    is1 = (ids == a1).astype(jnp.float32)
    is2 = (ids == a2).astype(jnp.float32)
    wfull_ref[...] = w1 * is1 + w2 * is2
    cnt_part = jnp.sum(is1 + is2, axis=0, keepdims=True)
    ex = jnp.exp(s - m1)
    sm = ex / jnp.sum(ex, axis=1, keepdims=True)
    imp_part = jnp.sum(sm, axis=0, keepdims=True)

    @pl.when(i == 0)
    def _():
        cnt_ref[...] = cnt_part
        imp_ref[...] = imp_part

    @pl.when(i > 0)
    def _():
        cnt_ref[...] += cnt_part
        imp_ref[...] += imp_part

    @pl.when(i == pl.num_programs(0) - 1)
    def _():
        c = cnt_ref[...]
        cm = jnp.sum(c) / E
        cvar = jnp.sum((c - cm) ** 2) / (E - 1)
        ll_ref[...] = cvar.reshape(1, 1) / (E * (B / E))
        im = imp_ref[...]
        imm = jnp.sum(im) / E
        ivar = jnp.sum((im - imm) ** 2) / (E - 1)
        il_ref[...] = ivar.reshape(1, 1) / (imm + 1e-8)


def _mlp_body(x_ref, w1_ref, b1_ref, w2_ref, b2_ref, wfull_ref, out_ref):
    x = x_ref[...].astype(jnp.bfloat16)
    wf = wfull_ref[...]
    acc = jnp.zeros((TM, D), jnp.float32)
    for e in range(E):
        h = jax.lax.dot_general(
            x, w1_ref[e], (((1,), (1,)), ((), ())),
            preferred_element_type=jnp.float32) + b1_ref[e][None, :]
        h = jnp.maximum(h, 0.0).astype(jnp.bfloat16)
        o = jax.lax.dot_general(
            h, w2_ref[e], (((1,), (1,)), ((), ())),
            preferred_element_type=jnp.float32) + b2_ref[e][None, :]
        acc = acc + o * wf[:, e:e + 1]
    out_ref[...] = acc


def kernel(x, Wg, bg, W1, b1, W2, b2):
    nt = B // TM
    wfull, cnt, imp, ll, il = pl.pallas_call(
        _routing_body,
        grid=(nt,),
        in_specs=[
            pl.BlockSpec((TM, D), lambda i: (i, 0)),
            pl.BlockSpec((E, D), lambda i: (0, 0)),
            pl.BlockSpec((1, E), lambda i: (0, 0)),
        ],
        out_specs=[
            pl.BlockSpec((TM, E), lambda i: (i, 0)),
            pl.BlockSpec((1, E), lambda i: (0, 0)),
            pl.BlockSpec((1, E), lambda i: (0, 0)),
            pl.BlockSpec((1, 1), lambda i: (0, 0)),
            pl.BlockSpec((1, 1), lambda i: (0, 0)),
        ],
        out_shape=[
            jax.ShapeDtypeStruct((B, E), jnp.float32),
            jax.ShapeDtypeStruct((1, E), jnp.float32),
            jax.ShapeDtypeStruct((1, E), jnp.float32),
            jax.ShapeDtypeStruct((1, 1), jnp.float32),
            jax.ShapeDtypeStruct((1, 1), jnp.float32),
        ],
    )(x, Wg, bg.reshape(1, E))

    w1b = W1.astype(jnp.bfloat16)
    w2b = W2.astype(jnp.bfloat16)
    out = pl.pallas_call(
        _mlp_body,
        grid=(nt,),
        in_specs=[
            pl.BlockSpec((TM, D), lambda i: (i, 0)),
            pl.BlockSpec((E, D, D), lambda i: (0, 0, 0)),
            pl.BlockSpec((E, D), lambda i: (0, 0)),
            pl.BlockSpec((E, D, D), lambda i: (0, 0, 0)),
            pl.BlockSpec((E, D), lambda i: (0, 0)),
            pl.BlockSpec((TM, E), lambda i: (i, 0)),
        ],
        out_specs=pl.BlockSpec((TM, D), lambda i: (i, 0)),
        out_shape=jax.ShapeDtypeStruct((B, D), jnp.float32),
    )(x, w1b, b1, w2b, b2, wfull)

    return out, ll.reshape(()), il.reshape(())
